# baseline (speedup 1.0000x reference)
import functools

import jax
import jax.numpy as jnp
from jax import lax
from jax.experimental import pallas as pl
from jax.experimental.pallas import tpu as pltpu
from jax.experimental.pallas import tpu_sc as plsc

NDCG_W = 0.4
LISTMLE_W = 0.3
BINARY_W = 0.3
K = 10
MARGIN = 0.1
LN2 = 0.6931471805599453

N_REAL = 20480
NROW = 256
NPAD = NROW * 128
FILLER = 0x7F800001


def _mangle(u):
    m = u >> 31
    return u ^ (m & 0x7FFFFFFF)


def _roll(x, shift, axis):
    return jnp.roll(x, shift, axis=axis)


def _body(p_ref, r_ref, a_ref, l_ref, out_ref):
    A = a_ref[...]
    Lab = l_ref[...]

    merged = jnp.where(Lab == 0, A, A - MARGIN)
    u = lax.bitcast_convert_type(merged, jnp.int32)
    s = _mangle(u)
    keys160 = (s & -2) | jnp.where(Lab == 1, 1, 0)
    x = jnp.concatenate(
        [keys160, jnp.full((NROW - 160, 128), FILLER, jnp.int32)], axis=0)

    iota_l = lax.broadcasted_iota(jnp.int32, (NROW, 128), 1)
    iota_r = lax.broadcasted_iota(jnp.int32, (NROW, 128), 0)

    P = p_ref[...]
    R = r_ref[...]
    D, B = P.shape
    idx = lax.broadcasted_iota(jnp.int32, (D, B), 0)
    rank_p = jnp.zeros((D, B), jnp.float32)
    rank_r = jnp.zeros((D, B), jnp.float32)
    s_exp = jnp.zeros((D, B), jnp.float32)

    def rank_step(j, rank_p, rank_r, s_exp):
        Pj = P[j:j + 1, :]
        Rj = R[j:j + 1, :]
        beats_p = (Pj > P) | ((Pj == P) & (j < idx))
        beats_r = (Rj > R) | ((Rj == R) & (j < idx))
        rank_p = rank_p + beats_p.astype(jnp.float32)
        rank_r = rank_r + beats_r.astype(jnp.float32)
        s_exp = s_exp + jnp.exp(Pj) * (1.0 - beats_r.astype(jnp.float32))
        return rank_p, rank_r, s_exp

    substeps = []
    for stage in range(1, 16):
        k = 1 << stage
        j = k >> 1
        while j >= 1:
            substeps.append((k, j))
            j >>= 1

    xs = [x[64 * ci:64 * ci + 64] for ci in range(4)]
    iota_lc = lax.broadcasted_iota(jnp.int32, (64, 128), 1)
    iota_rc = lax.broadcasted_iota(jnp.int32, (64, 128), 0)

    def chunk_up(k, ci):
        if k < 128:
            return (iota_lc & k) == 0
        if k >= NPAD:
            return True
        kr = k >> 7
        if kr < 64:
            return (iota_rc & kr) == 0
        return ((64 * ci) & kr) == 0

    def cx(a, p, want_max):
        if want_max is True:
            return jnp.maximum(a, p)
        if want_max is False:
            return jnp.minimum(a, p)
        return jnp.where(want_max, jnp.maximum(a, p), jnp.minimum(a, p))

    def xor_mask(up, lower):
        if up is True:
            return ~lower if not isinstance(lower, bool) else not lower
        if up is False:
            return lower
        return jnp.logical_xor(up, lower)

    next_rank = 0
    for si, (k, j) in enumerate(substeps):
        skip3 = k < 16384
        if j < 128:
            lower = (iota_lc & j) == 0
            for ci in range(3 if skip3 else 4):
                pm = _roll(xs[ci], -j, 1)
                pp = _roll(xs[ci], j, 1)
                p = jnp.where(lower, pm, pp)
                xs[ci] = cx(xs[ci], p, xor_mask(chunk_up(k, ci), lower))
        else:
            jr = j >> 7
            if jr < 64:
                lower = (iota_rc & jr) == 0
                for ci in range(3 if skip3 else 4):
                    pm = _roll(xs[ci], -jr, 0)
                    pp = _roll(xs[ci], jr, 0)
                    p = jnp.where(lower, pm, pp)
                    xs[ci] = cx(xs[ci], p, xor_mask(chunk_up(k, ci), lower))
            else:
                pairs = ((0, 1), (2, 3)) if jr == 64 else ((0, 2), (1, 3))
                for a, b in pairs:
                    up_a = chunk_up(k, a)
                    na = cx(xs[a], xs[b], xor_mask(up_a, True))
                    nb = cx(xs[b], xs[a], xor_mask(chunk_up(k, b), False))
                    xs[a], xs[b] = na, nb
        if si % 6 == 5 and next_rank < D:
            rank_p, rank_r, s_exp = rank_step(next_rank, rank_p, rank_r, s_exp)
            next_rank += 1
    while next_rank < D:
        rank_p, rank_r, s_exp = rank_step(next_rank, rank_p, rank_r, s_exp)
        next_rank += 1
    x = jnp.concatenate(xs, axis=0)

    pos_tag = (x & 1) == 1
    sk = x & -2
    v = lax.bitcast_convert_type(_mangle(sk), jnp.float32)
    flat = iota_r * 128 + iota_l
    real = flat < N_REAL
    negm = (~pos_tag) & real

    cnt = jnp.where(negm, 1.0, 0.0)
    val = jnp.where(negm, v, 0.0)

    for sh in (1, 2, 4, 8, 16, 32, 64):
        lm = iota_l >= sh
        cnt = cnt + jnp.where(lm, _roll(cnt, sh, 1), 0.0)
        val = val + jnp.where(lm, _roll(val, sh, 1), 0.0)
    rt_c = cnt[:, 127:128]
    rt_v = val[:, 127:128]
    ic = rt_c
    iv = rt_v
    iota_rc = iota_r[:, 0:1]
    for sh in (1, 2, 4, 8, 16, 32, 64, 128):
        rm = iota_rc >= sh
        ic = ic + jnp.where(rm, _roll(ic, sh, 0), 0.0)
        iv = iv + jnp.where(rm, _roll(iv, sh, 0), 0.0)
    cnt = cnt + (ic - rt_c)
    val = val + (iv - rt_v)
    n_neg_s = ic[NROW - 1:NROW, :]
    s_tot = iv[NROW - 1:NROW, :]

    contrib = jnp.where(pos_tag & real,
                        (s_tot - val) - v * (n_neg_s - cnt), 0.0)
    pair_sum = jnp.sum(contrib)

    w_p = jnp.where(rank_p < K, LN2 / jnp.log(rank_p + 2.0), 0.0)
    w_r = jnp.where(rank_r < K, LN2 / jnp.log(rank_r + 2.0), 0.0)
    dcg = jnp.sum(R * w_p, axis=0, keepdims=True)
    idcg = jnp.sum(R * w_r, axis=0, keepdims=True)
    ndcg_loss = 1.0 - jnp.sum(dcg / (idcg + 1e-8)) / B

    listmle = jnp.sum(jnp.log(s_exp + 1e-8) - P) / B

    pos = (Lab == 1).astype(jnp.float32)
    n_pos = jnp.sum(pos)
    n_neg = jnp.float32(N_REAL) - n_pos
    bce = jnp.sum(jnp.maximum(A, 0.0) - A * pos
                  + jnp.log(1.0 + jnp.exp(-jnp.abs(A)))) / N_REAL
    rank_loss = pair_sum / jnp.maximum(n_pos * n_neg, 1.0)
    bin_loss = jnp.where((n_pos > 0) & (n_neg > 0), rank_loss, bce)

    total = NDCG_W * ndcg_loss + LISTMLE_W * listmle + BINARY_W * bin_loss
    out_ref[...] = jnp.reshape(total, (1, 1))


_SC_MESH = plsc.VectorSubcoreMesh(core_axis_name="c", subcore_axis_name="s")


@functools.partial(
    pl.kernel,
    mesh=_SC_MESH,
    out_type=jax.ShapeDtypeStruct((16,), jnp.float32),
    scratch_types=[pltpu.VMEM((16,), jnp.float32)],
)
def _sc_probe(x_hbm, out_hbm, buf):
    c = lax.axis_index("c")
    s = lax.axis_index("s")

    @pl.when((c == 0) & (s == 0))
    def _():
        pltpu.sync_copy(x_hbm.at[pl.ds(0, 16)], buf)
        buf[...] = buf[...] + 1.0
        pltpu.sync_copy(buf, out_hbm)


def kernel(predictions, relevance_scores, labels):
    B, D = predictions.shape
    A = predictions.reshape(160, 128)
    L = labels.reshape(160, 128)
    PT = predictions.T
    RT = relevance_scores.T

    out = pl.pallas_call(
        _body,
        in_specs=[
            pl.BlockSpec((D, B), lambda: (0, 0)),
            pl.BlockSpec((D, B), lambda: (0, 0)),
            pl.BlockSpec((160, 128), lambda: (0, 0)),
            pl.BlockSpec((160, 128), lambda: (0, 0)),
        ],
        out_specs=pl.BlockSpec((1, 1), lambda: (0, 0)),
        out_shape=jax.ShapeDtypeStruct((1, 1), jnp.float32),
    )(PT, RT, A, L)
    sc_out = _sc_probe(predictions.reshape(-1))
    return out.reshape(()) + sc_out[0] * 0.0


# speedup vs baseline: 2.2286x; 2.2286x over previous
import jax
import jax.numpy as jnp
from jax import lax
from jax.experimental import pallas as pl
from jax.experimental.pallas import tpu as pltpu

NDCG_W = 0.4
LISTMLE_W = 0.3
BINARY_W = 0.3
K = 10
MARGIN = 0.1
LN2 = 0.6931471805599453

N_REAL = 20480
NROW = 256
NPAD = NROW * 128
FILLER = 0x7F800001


def _mangle(u):
    m = u >> 31
    return u ^ (m & 0x7FFFFFFF)


def _roll(x, shift, axis):
    return jnp.roll(x, shift, axis=axis)


def _body(p_ref, r_ref, a_ref, l_ref, out_ref):
    A = a_ref[...]
    Lab = l_ref[...]

    merged = jnp.where(Lab == 0, A, A - MARGIN)
    u = lax.bitcast_convert_type(merged, jnp.int32)
    s = _mangle(u)
    keys160 = (s & -2) | jnp.where(Lab == 1, 1, 0)
    x = jnp.concatenate(
        [keys160, jnp.full((NROW - 160, 128), FILLER, jnp.int32)], axis=0)

    iota_l = lax.broadcasted_iota(jnp.int32, (NROW, 128), 1)
    iota_r = lax.broadcasted_iota(jnp.int32, (NROW, 128), 0)

    P = p_ref[...]
    R = r_ref[...]
    D, B = P.shape
    idx = lax.broadcasted_iota(jnp.int32, (D, B), 0)
    rank_p = jnp.zeros((D, B), jnp.float32)
    rank_r = jnp.zeros((D, B), jnp.float32)
    s_exp = jnp.zeros((D, B), jnp.float32)

    def rank_step(j, rank_p, rank_r, s_exp):
        Pj = P[j:j + 1, :]
        Rj = R[j:j + 1, :]
        beats_p = (Pj > P) | ((Pj == P) & (j < idx))
        beats_r = (Rj > R) | ((Rj == R) & (j < idx))
        rank_p = rank_p + beats_p.astype(jnp.float32)
        rank_r = rank_r + beats_r.astype(jnp.float32)
        s_exp = s_exp + jnp.exp(Pj) * (1.0 - beats_r.astype(jnp.float32))
        return rank_p, rank_r, s_exp

    substeps = []
    for stage in range(1, 16):
        k = 1 << stage
        j = k >> 1
        while j >= 1:
            substeps.append((k, j))
            j >>= 1

    xs = [x[64 * ci:64 * ci + 64] for ci in range(4)]
    iota_lc = lax.broadcasted_iota(jnp.int32, (64, 128), 1)
    iota_rc = lax.broadcasted_iota(jnp.int32, (64, 128), 0)

    def chunk_up(k, ci):
        if k < 128:
            return (iota_lc & k) == 0
        if k >= NPAD:
            return True
        kr = k >> 7
        if kr < 64:
            return (iota_rc & kr) == 0
        return ((64 * ci) & kr) == 0

    def cx(a, p, want_max):
        if want_max is True:
            return jnp.maximum(a, p)
        if want_max is False:
            return jnp.minimum(a, p)
        return jnp.where(want_max, jnp.maximum(a, p), jnp.minimum(a, p))

    def xor_mask(up, lower):
        if up is True:
            return ~lower if not isinstance(lower, bool) else not lower
        if up is False:
            return lower
        return jnp.logical_xor(up, lower)

    next_rank = 0
    for si, (k, j) in enumerate(substeps):
        skip3 = k < 16384
        if j < 128:
            lower = (iota_lc & j) == 0
            for ci in range(3 if skip3 else 4):
                pm = _roll(xs[ci], -j, 1)
                pp = _roll(xs[ci], j, 1)
                p = jnp.where(lower, pm, pp)
                xs[ci] = cx(xs[ci], p, xor_mask(chunk_up(k, ci), lower))
        else:
            jr = j >> 7
            if jr < 64:
                lower = (iota_rc & jr) == 0
                for ci in range(3 if skip3 else 4):
                    pm = _roll(xs[ci], -jr, 0)
                    pp = _roll(xs[ci], jr, 0)
                    p = jnp.where(lower, pm, pp)
                    xs[ci] = cx(xs[ci], p, xor_mask(chunk_up(k, ci), lower))
            else:
                pairs = ((0, 1), (2, 3)) if jr == 64 else ((0, 2), (1, 3))
                for a, b in pairs:
                    up_a = chunk_up(k, a)
                    na = cx(xs[a], xs[b], xor_mask(up_a, True))
                    nb = cx(xs[b], xs[a], xor_mask(chunk_up(k, b), False))
                    xs[a], xs[b] = na, nb
        if si % 6 == 5 and next_rank < D:
            rank_p, rank_r, s_exp = rank_step(next_rank, rank_p, rank_r, s_exp)
            next_rank += 1
    while next_rank < D:
        rank_p, rank_r, s_exp = rank_step(next_rank, rank_p, rank_r, s_exp)
        next_rank += 1
    x = jnp.concatenate(xs, axis=0)

    pos_tag = (x & 1) == 1
    sk = x & -2
    v = lax.bitcast_convert_type(_mangle(sk), jnp.float32)
    flat = iota_r * 128 + iota_l
    real = flat < N_REAL
    negm = (~pos_tag) & real

    cnt = jnp.where(negm, 1.0, 0.0)
    val = jnp.where(negm, v, 0.0)

    for sh in (1, 2, 4, 8, 16, 32, 64):
        lm = iota_l >= sh
        cnt = cnt + jnp.where(lm, _roll(cnt, sh, 1), 0.0)
        val = val + jnp.where(lm, _roll(val, sh, 1), 0.0)
    rt_c = cnt[:, 127:128]
    rt_v = val[:, 127:128]
    ic = rt_c
    iv = rt_v
    iota_rc = iota_r[:, 0:1]
    for sh in (1, 2, 4, 8, 16, 32, 64, 128):
        rm = iota_rc >= sh
        ic = ic + jnp.where(rm, _roll(ic, sh, 0), 0.0)
        iv = iv + jnp.where(rm, _roll(iv, sh, 0), 0.0)
    cnt = cnt + (ic - rt_c)
    val = val + (iv - rt_v)
    n_neg_s = ic[NROW - 1:NROW, :]
    s_tot = iv[NROW - 1:NROW, :]

    contrib = jnp.where(pos_tag & real,
                        (s_tot - val) - v * (n_neg_s - cnt), 0.0)
    pair_sum = jnp.sum(contrib)

    w_p = jnp.where(rank_p < K, LN2 / jnp.log(rank_p + 2.0), 0.0)
    w_r = jnp.where(rank_r < K, LN2 / jnp.log(rank_r + 2.0), 0.0)
    dcg = jnp.sum(R * w_p, axis=0, keepdims=True)
    idcg = jnp.sum(R * w_r, axis=0, keepdims=True)
    ndcg_loss = 1.0 - jnp.sum(dcg / (idcg + 1e-8)) / B

    listmle = jnp.sum(jnp.log(s_exp + 1e-8) - P) / B

    pos = (Lab == 1).astype(jnp.float32)
    n_pos = jnp.sum(pos)
    n_neg = jnp.float32(N_REAL) - n_pos
    bce = jnp.sum(jnp.maximum(A, 0.0) - A * pos
                  + jnp.log(1.0 + jnp.exp(-jnp.abs(A)))) / N_REAL
    rank_loss = pair_sum / jnp.maximum(n_pos * n_neg, 1.0)
    bin_loss = jnp.where((n_pos > 0) & (n_neg > 0), rank_loss, bce)

    total = NDCG_W * ndcg_loss + LISTMLE_W * listmle + BINARY_W * bin_loss
    out_ref[...] = jnp.reshape(total, (1, 1))


def kernel(predictions, relevance_scores, labels):
    B, D = predictions.shape
    A = predictions.reshape(160, 128)
    L = labels.reshape(160, 128)
    PT = predictions.T
    RT = relevance_scores.T

    out = pl.pallas_call(
        _body,
        in_specs=[
            pl.BlockSpec((D, B), lambda: (0, 0)),
            pl.BlockSpec((D, B), lambda: (0, 0)),
            pl.BlockSpec((160, 128), lambda: (0, 0)),
            pl.BlockSpec((160, 128), lambda: (0, 0)),
        ],
        out_specs=pl.BlockSpec((1, 1), lambda: (0, 0)),
        out_shape=jax.ShapeDtypeStruct((1, 1), jnp.float32),
    )(PT, RT, A, L)
    return out.reshape(())


# speedup vs baseline: 2.9453x; 1.3216x over previous
import jax
import jax.numpy as jnp
from jax import lax
from jax.experimental import pallas as pl
from jax.experimental.pallas import tpu as pltpu

NDCG_W = 0.4
LISTMLE_W = 0.3
BINARY_W = 0.3
K = 10
MARGIN = 0.1
LN2 = 0.6931471805599453

N_REAL = 20480
NROW = 256
NPAD = NROW * 128
FILLER = 0x7F800001


def _mangle(u):
    m = u >> 31
    return u ^ (m & 0x7FFFFFFF)


def _roll(x, shift, axis):
    return jnp.roll(x, shift, axis=axis)


def _body(p_ref, r_ref, lt_ref, out_ref):
    PT = p_ref[...]
    LabT = lt_ref[...]

    merged = jnp.where(LabT == 0, PT, PT - MARGIN)
    u = lax.bitcast_convert_type(merged, jnp.int32)
    s = _mangle(u)
    keysT = (s & -2) | jnp.where(LabT == 1, 1, 0)
    keys160 = jnp.concatenate(
        [keysT[:, 128 * c:128 * (c + 1)] for c in range(8)], axis=0)
    x = jnp.concatenate(
        [keys160, jnp.full((NROW - 160, 128), FILLER, jnp.int32)], axis=0)

    iota_l = lax.broadcasted_iota(jnp.int32, (NROW, 128), 1)
    iota_r = lax.broadcasted_iota(jnp.int32, (NROW, 128), 0)

    P = p_ref[...]
    R = r_ref[...]
    D, B = P.shape
    idx = lax.broadcasted_iota(jnp.int32, (D, B), 0)
    rank_p = jnp.zeros((D, B), jnp.float32)
    rank_r = jnp.zeros((D, B), jnp.float32)
    s_exp = jnp.zeros((D, B), jnp.float32)

    def rank_step(j, rank_p, rank_r, s_exp):
        Pj = P[j:j + 1, :]
        Rj = R[j:j + 1, :]
        beats_p = (Pj > P) | ((Pj == P) & (j < idx))
        beats_r = (Rj > R) | ((Rj == R) & (j < idx))
        rank_p = rank_p + beats_p.astype(jnp.float32)
        rank_r = rank_r + beats_r.astype(jnp.float32)
        s_exp = s_exp + jnp.exp(Pj) * (1.0 - beats_r.astype(jnp.float32))
        return rank_p, rank_r, s_exp

    substeps = []
    for stage in range(1, 16):
        k = 1 << stage
        j = k >> 1
        while j >= 1:
            substeps.append((k, j))
            j >>= 1

    xs = [x[64 * ci:64 * ci + 64] for ci in range(4)]
    iota_lc = lax.broadcasted_iota(jnp.int32, (64, 128), 1)
    iota_rc = lax.broadcasted_iota(jnp.int32, (64, 128), 0)

    def chunk_up(k, ci):
        if k < 128:
            return (iota_lc & k) == 0
        if k >= NPAD:
            return True
        kr = k >> 7
        if kr < 64:
            return (iota_rc & kr) == 0
        return ((64 * ci) & kr) == 0

    def cx(a, p, want_max):
        if want_max is True:
            return jnp.maximum(a, p)
        if want_max is False:
            return jnp.minimum(a, p)
        return jnp.where(want_max, jnp.maximum(a, p), jnp.minimum(a, p))

    def xor_mask(up, lower):
        if up is True:
            return ~lower if not isinstance(lower, bool) else not lower
        if up is False:
            return lower
        return jnp.logical_xor(up, lower)

    next_rank = 0
    for si, (k, j) in enumerate(substeps):
        skip3 = k < 16384
        if j < 128:
            lower = (iota_lc & j) == 0
            for ci in range(3 if skip3 else 4):
                pm = _roll(xs[ci], -j, 1)
                pp = _roll(xs[ci], j, 1)
                p = jnp.where(lower, pm, pp)
                xs[ci] = cx(xs[ci], p, xor_mask(chunk_up(k, ci), lower))
        else:
            jr = j >> 7
            if jr < 64:
                lower = (iota_rc & jr) == 0
                for ci in range(3 if skip3 else 4):
                    pm = _roll(xs[ci], -jr, 0)
                    pp = _roll(xs[ci], jr, 0)
                    p = jnp.where(lower, pm, pp)
                    xs[ci] = cx(xs[ci], p, xor_mask(chunk_up(k, ci), lower))
            else:
                pairs = ((0, 1), (2, 3)) if jr == 64 else ((0, 2), (1, 3))
                for a, b in pairs:
                    up_a = chunk_up(k, a)
                    na = cx(xs[a], xs[b], xor_mask(up_a, True))
                    nb = cx(xs[b], xs[a], xor_mask(chunk_up(k, b), False))
                    xs[a], xs[b] = na, nb
        if si % 6 == 5 and next_rank < D:
            rank_p, rank_r, s_exp = rank_step(next_rank, rank_p, rank_r, s_exp)
            next_rank += 1
    while next_rank < D:
        rank_p, rank_r, s_exp = rank_step(next_rank, rank_p, rank_r, s_exp)
        next_rank += 1
    x = jnp.concatenate(xs, axis=0)

    pos_tag = (x & 1) == 1
    sk = x & -2
    v = lax.bitcast_convert_type(_mangle(sk), jnp.float32)
    flat = iota_r * 128 + iota_l
    real = flat < N_REAL
    negm = (~pos_tag) & real

    cnt = jnp.where(negm, 1.0, 0.0)
    val = jnp.where(negm, v, 0.0)

    for sh in (1, 2, 4, 8, 16, 32, 64):
        lm = iota_l >= sh
        cnt = cnt + jnp.where(lm, _roll(cnt, sh, 1), 0.0)
        val = val + jnp.where(lm, _roll(val, sh, 1), 0.0)
    rt_c = cnt[:, 127:128]
    rt_v = val[:, 127:128]
    ic = rt_c
    iv = rt_v
    iota_rc = iota_r[:, 0:1]
    for sh in (1, 2, 4, 8, 16, 32, 64, 128):
        rm = iota_rc >= sh
        ic = ic + jnp.where(rm, _roll(ic, sh, 0), 0.0)
        iv = iv + jnp.where(rm, _roll(iv, sh, 0), 0.0)
    cnt = cnt + (ic - rt_c)
    val = val + (iv - rt_v)
    n_neg_s = ic[NROW - 1:NROW, :]
    s_tot = iv[NROW - 1:NROW, :]

    contrib = jnp.where(pos_tag & real,
                        (s_tot - val) - v * (n_neg_s - cnt), 0.0)
    pair_sum = jnp.sum(contrib)

    w_p = jnp.where(rank_p < K, LN2 / jnp.log(rank_p + 2.0), 0.0)
    w_r = jnp.where(rank_r < K, LN2 / jnp.log(rank_r + 2.0), 0.0)
    dcg = jnp.sum(R * w_p, axis=0, keepdims=True)
    idcg = jnp.sum(R * w_r, axis=0, keepdims=True)
    ndcg_loss = 1.0 - jnp.sum(dcg / (idcg + 1e-8)) / B

    listmle = jnp.sum(jnp.log(s_exp + 1e-8) - P) / B

    pos = (LabT == 1).astype(jnp.float32)
    n_pos = jnp.sum(pos)
    n_neg = jnp.float32(N_REAL) - n_pos
    bce = jnp.sum(jnp.maximum(PT, 0.0) - PT * pos
                  + jnp.log(1.0 + jnp.exp(-jnp.abs(PT)))) / N_REAL
    rank_loss = pair_sum / jnp.maximum(n_pos * n_neg, 1.0)
    bin_loss = jnp.where((n_pos > 0) & (n_neg > 0), rank_loss, bce)

    total = NDCG_W * ndcg_loss + LISTMLE_W * listmle + BINARY_W * bin_loss
    out_ref[...] = jnp.reshape(total, (1, 1))


def kernel(predictions, relevance_scores, labels):
    B, D = predictions.shape
    PT = predictions.T
    RT = relevance_scores.T
    LT = labels.T

    out = pl.pallas_call(
        _body,
        in_specs=[
            pl.BlockSpec((D, B), lambda: (0, 0)),
            pl.BlockSpec((D, B), lambda: (0, 0)),
            pl.BlockSpec((D, B), lambda: (0, 0)),
        ],
        out_specs=pl.BlockSpec((1, 1), lambda: (0, 0)),
        out_shape=jax.ShapeDtypeStruct((1, 1), jnp.float32),
    )(PT, RT, LT)
    return out.reshape(())
